# Initial kernel scaffold; baseline (speedup 1.0000x reference)
import jax
import jax.numpy as jnp
from jax.experimental import pallas as pl


def kernel(original, posterior, prior, beta_ce, beta_cu, cu_mult, ce_offset):
    raise NotImplementedError("write your pallas kernel here")


# speedup vs baseline: 1.0076x; 1.0076x over previous
import functools

import jax
import jax.numpy as jnp
from jax import lax
from jax.experimental import pallas as pl
from jax.experimental.pallas import tpu as pltpu

_CAPACITY = 0.5
_WINDOW = 100


def _dist_kernel(o_ref, p_ref, r_ref, dst_ref, dch_ref):
    o = o_ref[0]
    p = p_ref[0]
    r = r_ref[0]
    dst_ref[0, 0, 0, :] = jnp.mean((p - o) ** 2, axis=-1)
    dch_ref[0, 0, 0, :] = jnp.mean((p - r) ** 2, axis=-1)


def _route_kernel(dst_ref, dch_ref, par_ref, mask_ref, sig_ref, sce_ref,
                  scu_ref, *, target):
    d_st = dst_ref[...]
    d_ch = dch_ref[...]
    sp_ce = par_ref[0]
    sp_cu = par_ref[1]
    cu_mult = par_ref[2]
    log_off = par_ref[3]
    B, S = d_st.shape
    w = _WINDOW
    padded = jnp.concatenate(
        [jnp.broadcast_to(d_st[:, :1], (B, w - 1)), d_st], axis=1)
    c = padded
    sh = 1
    while sh < padded.shape[1]:
        z = jnp.zeros((B, sh), c.dtype)
        c = c + jnp.concatenate([z, c[:, :-sh]], axis=1)
        sh *= 2
    c = jnp.concatenate([jnp.zeros((B, 1), c.dtype), c], axis=1)
    ma = (c[:, w:] - c[:, :-w]) / jnp.float32(w)

    ce_val = d_st - (d_ch - log_off)
    cu_val = d_st - cu_mult * ma
    s_ce = 1.0 / (1.0 + jnp.exp(-(sp_ce * ce_val)))
    s_cu = 1.0 / (1.0 + jnp.exp(-(sp_cu * cu_val)))
    signal = s_ce + s_cu - s_ce * s_cu

    if target is None:
        mask_ref[...] = jnp.ones_like(signal)
    else:
        u = lax.bitcast_convert_type(signal, jnp.int32)
        imin = jnp.int32(-(2 ** 31))
        ks = jnp.where(u >= 0, u, (~u) ^ imin)
        ans = jnp.int32(0)
        for b in range(31, -1, -1):
            bit = imin if b == 31 else jnp.int32(1 << b)
            cand = ans | bit
            cnt = jnp.sum((ks < (cand ^ imin)).astype(jnp.int32))
            ans = jnp.where(cnt <= target, cand, ans)
        tbits = jnp.where(ans < 0, ans ^ imin, ~ans)
        thr = lax.bitcast_convert_type(tbits, jnp.float32)
        mask_ref[...] = (signal >= thr).astype(jnp.float32)
    sig_ref[...] = signal
    sce_ref[...] = s_ce
    scu_ref[...] = s_cu


def kernel(original, posterior, prior, beta_ce, beta_cu, cu_mult, ce_offset):
    B, S, D = original.shape
    SB = 512
    NS = S // SB
    dst4, dch4 = pl.pallas_call(
        _dist_kernel,
        grid=(B, NS),
        in_specs=[pl.BlockSpec((1, SB, D), lambda b, s: (b, s, 0))] * 3,
        out_specs=[pl.BlockSpec((1, 1, 1, SB), lambda b, s: (b, s, 0, 0))] * 2,
        out_shape=[jax.ShapeDtypeStruct((B, NS, 1, SB), jnp.float32)] * 2,
    )(original, posterior, prior)
    d_st = dst4.reshape(B, S)
    d_ch = dch4.reshape(B, S)

    params = jnp.stack([
        jax.nn.softplus(jnp.asarray(beta_ce, jnp.float32)),
        jax.nn.softplus(jnp.asarray(beta_cu, jnp.float32)),
        jnp.asarray(cu_mult, jnp.float32),
        jnp.log(jnp.asarray(ce_offset, jnp.float32) + 1e-10),
    ])

    n = B * S
    k = int(_CAPACITY * n)
    target = (n - k) if k < n else None
    mask, signal, s_ce, s_cu = pl.pallas_call(
        functools.partial(_route_kernel, target=target),
        in_specs=[
            pl.BlockSpec((B, S), lambda: (0, 0)),
            pl.BlockSpec((B, S), lambda: (0, 0)),
            pl.BlockSpec(memory_space=pltpu.SMEM),
        ],
        out_specs=[pl.BlockSpec((B, S), lambda: (0, 0))] * 4,
        out_shape=[jax.ShapeDtypeStruct((B, S), jnp.float32)] * 4,
    )(d_st, d_ch, params)
    return mask, signal, s_ce, s_cu
